# Initial kernel scaffold; baseline (speedup 1.0000x reference)
import jax
import jax.numpy as jnp
from jax.experimental import pallas as pl


def kernel(idxes, x, raw_weights):
    raise NotImplementedError("write your pallas kernel here")


# speedup vs baseline: 6.0734x; 6.0734x over previous
import jax
import jax.numpy as jnp
from jax import lax
from jax.experimental import pallas as pl
from jax.experimental.pallas import tpu as pltpu
from jax.experimental.pallas import tpu_sc as plsc

_BATCH = 16384
_NUM_DOMAINS = 100
_LANES = 16
_NUM_CORES = 2
_NUM_SUBCORES = 16
_NUM_WORKERS = _NUM_CORES * _NUM_SUBCORES
_CHUNK = _BATCH // _NUM_WORKERS
_WPAD = 128


def _sc_body(idx_hbm, x_hbm, w_hbm, out_hbm, idx_v, x_v, w_v, out_v):
    wid = lax.axis_index("s") * _NUM_CORES + lax.axis_index("c")
    base = wid * _CHUNK
    pltpu.sync_copy(w_hbm, w_v)
    pltpu.sync_copy(idx_hbm.at[pl.ds(base, _CHUNK)], idx_v)
    pltpu.sync_copy(x_hbm.at[pl.ds(base, _CHUNK)], x_v)
    zero = jnp.zeros((_LANES,), jnp.float32)
    for j in range(_WPAD // _LANES):
        sl = pl.ds(j * _LANES, _LANES)
        w_v[sl] = jnp.maximum(w_v[sl], zero)
    for i in range(_CHUNK // _LANES):
        sl = pl.ds(i * _LANES, _LANES)
        w_vec = plsc.load_gather(w_v, [idx_v[sl]])
        out_v[sl] = x_v[sl] * w_vec
    pltpu.sync_copy(out_v, out_hbm.at[pl.ds(base, _CHUNK)])


def kernel(idxes, x, raw_weights):
    idxes = idxes.astype(jnp.int32)
    x_flat = x.reshape(-1)
    w_pad = jnp.pad(raw_weights, (0, _WPAD - _NUM_DOMAINS))
    mesh = plsc.VectorSubcoreMesh(core_axis_name="c", subcore_axis_name="s")
    out = pl.kernel(
        _sc_body,
        out_type=jax.ShapeDtypeStruct((_BATCH,), jnp.float32),
        mesh=mesh,
        compiler_params=pltpu.CompilerParams(needs_layout_passes=False),
        scratch_types=[
            pltpu.VMEM((_CHUNK,), jnp.int32),
            pltpu.VMEM((_CHUNK,), jnp.float32),
            pltpu.VMEM((_WPAD,), jnp.float32),
            pltpu.VMEM((_CHUNK,), jnp.float32),
        ],
    )(idxes, x_flat, w_pad)
    return out.reshape(_BATCH, 1)


# speedup vs baseline: 6.2973x; 1.0369x over previous
import jax
import jax.numpy as jnp
from jax import lax
from jax.experimental import pallas as pl
from jax.experimental.pallas import tpu as pltpu
from jax.experimental.pallas import tpu_sc as plsc

_BATCH = 16384
_NUM_DOMAINS = 100
_LANES = 16
_NUM_CORES = 2
_NUM_SUBCORES = 16
_NUM_WORKERS = _NUM_CORES * _NUM_SUBCORES
_CHUNK = _BATCH // _NUM_WORKERS
_WPAD = 128


def _sc_body(idx_hbm, x_hbm, w_hbm, out_hbm, idx_v, x_v, w_v, out_v, sem):
    wid = lax.axis_index("s") * _NUM_CORES + lax.axis_index("c")
    base = wid * _CHUNK
    cw = pltpu.async_copy(w_hbm, w_v, sem)
    ci = pltpu.async_copy(idx_hbm.at[pl.ds(base, _CHUNK)], idx_v, sem)
    cx = pltpu.async_copy(x_hbm.at[pl.ds(base, _CHUNK)], x_v, sem)
    cw.wait()
    zero = jnp.zeros((_LANES,), jnp.float32)
    for j in range(_WPAD // _LANES):
        sl = pl.ds(j * _LANES, _LANES)
        w_v[sl] = jnp.maximum(w_v[sl], zero)
    ci.wait()
    cx.wait()
    for i in range(_CHUNK // _LANES):
        sl = pl.ds(i * _LANES, _LANES)
        w_vec = plsc.load_gather(w_v, [idx_v[sl]])
        out_v[sl] = x_v[sl] * w_vec
    pltpu.sync_copy(out_v, out_hbm.at[pl.ds(base, _CHUNK)])


def kernel(idxes, x, raw_weights):
    idxes = idxes.astype(jnp.int32)
    x_flat = x.reshape(-1)
    w_pad = jnp.pad(raw_weights, (0, _WPAD - _NUM_DOMAINS))
    mesh = plsc.VectorSubcoreMesh(core_axis_name="c", subcore_axis_name="s")
    out = pl.kernel(
        _sc_body,
        out_type=jax.ShapeDtypeStruct((_BATCH,), jnp.float32),
        mesh=mesh,
        compiler_params=pltpu.CompilerParams(needs_layout_passes=False),
        scratch_types=[
            pltpu.VMEM((_CHUNK,), jnp.int32),
            pltpu.VMEM((_CHUNK,), jnp.float32),
            pltpu.VMEM((_WPAD,), jnp.float32),
            pltpu.VMEM((_CHUNK,), jnp.float32),
            pltpu.SemaphoreType.DMA,
        ],
    )(idxes, x_flat, w_pad)
    return out.reshape(_BATCH, 1)


# speedup vs baseline: 6.3595x; 1.0099x over previous
import jax
import jax.numpy as jnp
from jax import lax
from jax.experimental import pallas as pl
from jax.experimental.pallas import tpu as pltpu
from jax.experimental.pallas import tpu_sc as plsc

_BATCH = 16384
_NUM_DOMAINS = 100
_LANES = 16
_NUM_CORES = 2
_NUM_SUBCORES = 16
_NUM_WORKERS = _NUM_CORES * _NUM_SUBCORES
_CHUNK = _BATCH // _NUM_WORKERS


def _sc_body(idx_hbm, x_hbm, w_hbm, out_hbm, idx_v, x_v, w_v, out_v, sem):
    wid = lax.axis_index("s") * _NUM_CORES + lax.axis_index("c")
    base = wid * _CHUNK
    cw = pltpu.async_copy(w_hbm, w_v, sem)
    ci = pltpu.async_copy(idx_hbm.at[pl.ds(base, _CHUNK)], idx_v, sem)
    cx = pltpu.async_copy(x_hbm.at[pl.ds(base, _CHUNK)], x_v, sem)
    cw.wait()
    ci.wait()
    cx.wait()
    zero = jnp.zeros((_LANES,), jnp.float32)
    for i in range(_CHUNK // _LANES):
        sl = pl.ds(i * _LANES, _LANES)
        w_vec = plsc.load_gather(w_v, [idx_v[sl]])
        out_v[sl] = x_v[sl] * jnp.maximum(w_vec, zero)
    pltpu.sync_copy(out_v, out_hbm.at[pl.ds(base, _CHUNK)])


def kernel(idxes, x, raw_weights):
    idxes = idxes.astype(jnp.int32)
    x_flat = x.reshape(-1)
    mesh = plsc.VectorSubcoreMesh(core_axis_name="c", subcore_axis_name="s")
    out = pl.kernel(
        _sc_body,
        out_type=jax.ShapeDtypeStruct((_BATCH,), jnp.float32),
        mesh=mesh,
        compiler_params=pltpu.CompilerParams(
            needs_layout_passes=False, use_tc_tiling_on_sc=False
        ),
        scratch_types=[
            pltpu.VMEM((_CHUNK,), jnp.int32),
            pltpu.VMEM((_CHUNK,), jnp.float32),
            pltpu.VMEM((_NUM_DOMAINS,), jnp.float32),
            pltpu.VMEM((_CHUNK,), jnp.float32),
            pltpu.SemaphoreType.DMA,
        ],
    )(idxes, x_flat, raw_weights)
    return out.reshape(_BATCH, 1)
